# baseline (speedup 1.0000x reference)
import functools

import jax
import jax.numpy as jnp
from jax import lax
from jax.experimental import pallas as pl
from jax.experimental.pallas import tpu as pltpu
from jax.experimental.pallas import tpu_sc as plsc


@functools.lru_cache(maxsize=None)
def _make_detiler(vocab: int, dim: int):
    info = plsc.get_sparse_core_info()
    num_cores, num_subcores = info.num_cores, info.num_subcores
    num_workers = num_cores * num_subcores
    lanes = 16
    bw = 128
    full = vocab // bw
    rem = vocab - full * bw
    per_w = full // num_workers
    extra = full - per_w * num_workers

    mesh = plsc.VectorSubcoreMesh(core_axis_name="c", subcore_axis_name="s")

    @functools.partial(
        pl.kernel,
        out_type=jax.ShapeDtypeStruct((vocab * dim,), jnp.float32),
        mesh=mesh,
        scratch_types=[
            pltpu.VMEM((dim, bw), jnp.float32),
            pltpu.VMEM((dim, bw), jnp.float32),
            pltpu.VMEM((bw * dim,), jnp.float32),
            pltpu.VMEM((bw * dim,), jnp.float32),
            pltpu.SemaphoreType.DMA,
            pltpu.SemaphoreType.DMA,
            pltpu.SemaphoreType.DMA,
            pltpu.SemaphoreType.DMA,
        ],
        compiler_params=pltpu.CompilerParams(
            use_tc_tiling_on_sc=True, needs_layout_passes=False
        ),
    )
    def detile_kernel(tt_hbm, tail_hbm, out_hbm, inb0, inb1, ob0, ob1, g0, g1, p0, p1):
        wid = lax.axis_index("s") * num_cores + lax.axis_index("c")
        count = per_w + jnp.where(wid < extra, 1, 0)
        start = wid * per_w + jnp.minimum(wid, extra)

        iota = lax.iota(jnp.int32, lanes)
        iota_d = iota * dim

        def load_block(blk, buf, sem):
            return pltpu.async_copy(
                tt_hbm.at[:, pl.ds(blk * bw, bw)], buf, sem
            )

        def wait_in(sem, buf):
            pltpu.make_async_copy(tt_hbm.at[:, pl.ds(0, bw)], buf, sem).wait()

        def wait_out(ob, sem):
            pltpu.make_async_copy(
                out_hbm.at[pl.ds(0, bw * dim)], ob, sem
            ).wait()

        def transpose_store(buf, ob, blk, psem):
            @plsc.parallel_loop(0, bw // lanes, unroll=2)
            def t_body(g):
                base = g * (lanes * dim)
                for d in range(dim):
                    v = buf[d, pl.ds(g * lanes, lanes)]
                    plsc.store_scatter(ob, [iota_d + (base + d)], v)

            return pltpu.async_copy(
                ob, out_hbm.at[pl.ds(blk * bw * dim, bw * dim)], psem
            )

        load_block(start, inb0, g0)

        def body(j, carry):
            k0 = 2 * j
            k1 = 2 * j + 1

            @pl.when(k1 < count)
            def _():
                load_block(start + k1, inb1, g1)

            @pl.when(k0 < count)
            def _():
                wait_in(g0, inb0)

                @pl.when(j > 0)
                def _():
                    wait_out(ob0, p0)

                transpose_store(inb0, ob0, start + k0, p0)

                @pl.when(k0 + 2 < count)
                def _():
                    load_block(start + k0 + 2, inb0, g0)

            @pl.when(k1 < count)
            def _():
                wait_in(g1, inb1)

                @pl.when(j > 0)
                def _():
                    wait_out(ob1, p1)

                transpose_store(inb1, ob1, start + k1, p1)

            return carry

        lax.fori_loop(0, (per_w + 2) // 2, body, 0)
        wait_out(ob0, p0)
        wait_out(ob1, p1)

        if rem:
            @pl.when(wid == num_workers - 1)
            def _():
                pltpu.sync_copy(tail_hbm, inb0)

                @plsc.parallel_loop(0, bw // lanes, unroll=2)
                def r_body(g):
                    base = g * (lanes * dim)
                    for d in range(dim):
                        v = inb0[d, pl.ds(g * lanes, lanes)]
                        plsc.store_scatter(ob0, [iota_d + (base + d)], v)

                pltpu.sync_copy(
                    ob0, out_hbm.at[pl.ds((vocab - bw) * dim, bw * dim)]
                )

    return detile_kernel


@functools.lru_cache(maxsize=None)
def _make_kernel(batch: int, fields: int, vocab: int, dim: int):
    info = plsc.get_sparse_core_info()
    num_cores, num_subcores = info.num_cores, info.num_subcores
    num_workers = num_cores * num_subcores
    lanes = 16
    bt = 128
    n_btiles = batch // bt
    ct_per_w = n_btiles // num_workers
    octets = dim // 8
    groups = bt // lanes
    n_out = batch * fields * dim

    mesh = plsc.VectorSubcoreMesh(core_axis_name="c", subcore_axis_name="s")

    @functools.partial(
        pl.kernel,
        out_type=jax.ShapeDtypeStruct((n_out,), jnp.float32),
        mesh=mesh,
        scratch_types=[
            pltpu.VMEM((fields, ct_per_w * bt), jnp.int32),
            pltpu.VMEM((ct_per_w, bt, dim), jnp.float32),
            pltpu.VMEM((ct_per_w, octets * 8 * bt), jnp.float32),
            pltpu.SemaphoreType.DMA,
        ]
        + [pltpu.SemaphoreType.DMA for _ in range(2 * ct_per_w)],
        compiler_params=pltpu.CompilerParams(
            use_tc_tiling_on_sc=False, needs_layout_passes=False
        ),
    )
    def emb_kernel(idx_hbm, table_hbm, out_hbm, idx_v, rows_v, ostage, sem_i, *sems):
        gsems, psems = sems[:ct_per_w], sems[ct_per_w:]
        wid = lax.axis_index("s") * num_cores + lax.axis_index("c")
        cb = wid * ct_per_w

        pltpu.sync_copy(
            idx_hbm.at[:, pl.ds(cb * bt, ct_per_w * bt)], idx_v
        )

        iota = lax.iota(jnp.int32, lanes)
        cvs = [
            ((iota + k * lanes) // 8) * (8 * bt) + ((iota + k * lanes) % 8) * bt
            for k in range(dim // lanes)
        ]

        def body(f, carry):
            gd = []
            for cc in range(ct_per_w):
                gd.append(
                    pltpu.async_copy(
                        table_hbm.at[idx_v.at[f, pl.ds(cc * bt, bt)]],
                        rows_v.at[cc],
                        gsems[cc],
                    )
                )
            pd = []
            for cc in range(ct_per_w):
                gd[cc].wait()

                @plsc.parallel_loop(0, bt, unroll=4)
                def transpose_body(b, _cc=cc):
                    for k in range(dim // lanes):
                        v = rows_v[_cc, b, pl.ds(k * lanes, lanes)]
                        plsc.store_scatter(ostage.at[_cc], [cvs[k] + b], v)

                for r in range(octets):
                    off = (
                        f * (octets * n_btiles * 8 * bt)
                        + r * (n_btiles * 8 * bt)
                        + (cb + cc) * (8 * bt)
                    )
                    pd.append(
                        pltpu.async_copy(
                            ostage.at[cc, pl.ds(r * 8 * bt, 8 * bt)],
                            out_hbm.at[pl.ds(off, 8 * bt)],
                            psems[cc],
                        )
                    )
            for d in pd:
                d.wait()
            return carry

        lax.fori_loop(0, fields, body, 0)

    return emb_kernel


def kernel(inputs, table):
    batch, fields = inputs.shape
    vocab, dim = table.shape
    idx_t = inputs.T.astype(jnp.int32)
    detile = _make_detiler(vocab, dim)
    table_t = table.T
    table_rm = detile(table_t, table_t[:, vocab - 128 :]).reshape(vocab, dim)
    emb = _make_kernel(batch, fields, vocab, dim)
    out1d = emb(idx_t, table_rm)
    t5 = out1d.reshape(fields, dim // 8, batch // 128, 8, 128)
    return t5.transpose(2, 4, 0, 1, 3).reshape(batch, fields, dim)


# speedup vs baseline: 1.1132x; 1.1132x over previous
import functools

import jax
import jax.numpy as jnp
from jax import lax
from jax.experimental import pallas as pl
from jax.experimental.pallas import tpu as pltpu
from jax.experimental.pallas import tpu_sc as plsc


@functools.lru_cache(maxsize=None)
def _make_detiler(vocab: int, dim: int):
    info = plsc.get_sparse_core_info()
    num_cores, num_subcores = info.num_cores, info.num_subcores
    num_workers = num_cores * num_subcores
    lanes = 16
    bw = 128
    full = vocab // bw
    rem = vocab - full * bw
    per_w = full // num_workers
    extra = full - per_w * num_workers

    mesh = plsc.VectorSubcoreMesh(core_axis_name="c", subcore_axis_name="s")

    @functools.partial(
        pl.kernel,
        out_type=jax.ShapeDtypeStruct((vocab * dim,), jnp.float32),
        mesh=mesh,
        scratch_types=[
            pltpu.VMEM((dim, bw), jnp.float32),
            pltpu.VMEM((dim, bw), jnp.float32),
            pltpu.VMEM((bw * dim,), jnp.float32),
            pltpu.VMEM((bw * dim,), jnp.float32),
            pltpu.SemaphoreType.DMA,
            pltpu.SemaphoreType.DMA,
            pltpu.SemaphoreType.DMA,
            pltpu.SemaphoreType.DMA,
        ],
        compiler_params=pltpu.CompilerParams(
            use_tc_tiling_on_sc=True, needs_layout_passes=False
        ),
    )
    def detile_kernel(tt_hbm, tail_hbm, out_hbm, inb0, inb1, ob0, ob1, g0, g1, p0, p1):
        wid = lax.axis_index("s") * num_cores + lax.axis_index("c")
        count = per_w + jnp.where(wid < extra, 1, 0)
        start = wid * per_w + jnp.minimum(wid, extra)

        iota = lax.iota(jnp.int32, lanes)
        iota_d = iota * dim

        def load_block(blk, buf, sem):
            return pltpu.async_copy(
                tt_hbm.at[:, pl.ds(blk * bw, bw)], buf, sem
            )

        def wait_in(sem, buf):
            pltpu.make_async_copy(tt_hbm.at[:, pl.ds(0, bw)], buf, sem).wait()

        def wait_out(ob, sem):
            pltpu.make_async_copy(
                out_hbm.at[pl.ds(0, bw * dim)], ob, sem
            ).wait()

        def transpose_store(buf, ob, blk, psem):
            @plsc.parallel_loop(0, (bw // lanes) * dim, unroll=16)
            def t_body(i):
                g = i // dim
                d = i % dim
                v = buf[d, pl.ds(g * lanes, lanes)]
                plsc.store_scatter(ob, [iota_d + (g * lanes * dim + d)], v)

            return pltpu.async_copy(
                ob, out_hbm.at[pl.ds(blk * bw * dim, bw * dim)], psem
            )

        load_block(start, inb0, g0)

        def body(j, carry):
            k0 = 2 * j
            k1 = 2 * j + 1

            @pl.when(k1 < count)
            def _():
                load_block(start + k1, inb1, g1)

            @pl.when(k0 < count)
            def _():
                wait_in(g0, inb0)

                @pl.when(j > 0)
                def _():
                    wait_out(ob0, p0)

                transpose_store(inb0, ob0, start + k0, p0)

                @pl.when(k0 + 2 < count)
                def _():
                    load_block(start + k0 + 2, inb0, g0)

            @pl.when(k1 < count)
            def _():
                wait_in(g1, inb1)

                @pl.when(j > 0)
                def _():
                    wait_out(ob1, p1)

                transpose_store(inb1, ob1, start + k1, p1)

            return carry

        lax.fori_loop(0, (per_w + 2) // 2, body, 0)
        wait_out(ob0, p0)
        wait_out(ob1, p1)

        if rem:
            @pl.when(wid == num_workers - 1)
            def _():
                pltpu.sync_copy(tail_hbm, inb0)

                @plsc.parallel_loop(0, bw // lanes, unroll=2)
                def r_body(g):
                    base = g * (lanes * dim)
                    for d in range(dim):
                        v = inb0[d, pl.ds(g * lanes, lanes)]
                        plsc.store_scatter(ob0, [iota_d + (base + d)], v)

                pltpu.sync_copy(
                    ob0, out_hbm.at[pl.ds((vocab - bw) * dim, bw * dim)]
                )

    return detile_kernel


@functools.lru_cache(maxsize=None)
def _make_kernel(batch: int, fields: int, vocab: int, dim: int):
    info = plsc.get_sparse_core_info()
    num_cores, num_subcores = info.num_cores, info.num_subcores
    num_workers = num_cores * num_subcores
    lanes = 16
    bt = 128
    n_btiles = batch // bt
    ct_per_w = n_btiles // num_workers
    octets = dim // 8
    groups = bt // lanes
    n_out = batch * fields * dim

    mesh = plsc.VectorSubcoreMesh(core_axis_name="c", subcore_axis_name="s")

    @functools.partial(
        pl.kernel,
        out_type=jax.ShapeDtypeStruct((n_out,), jnp.float32),
        mesh=mesh,
        scratch_types=[
            pltpu.VMEM((fields, ct_per_w * bt), jnp.int32),
            pltpu.VMEM((ct_per_w, bt, dim), jnp.float32),
            pltpu.VMEM((ct_per_w, octets * 8 * bt), jnp.float32),
            pltpu.SemaphoreType.DMA,
        ]
        + [pltpu.SemaphoreType.DMA for _ in range(2 * ct_per_w)],
        compiler_params=pltpu.CompilerParams(
            use_tc_tiling_on_sc=False, needs_layout_passes=False
        ),
    )
    def emb_kernel(idx_hbm, table_hbm, out_hbm, idx_v, rows_v, ostage, sem_i, *sems):
        gsems, psems = sems[:ct_per_w], sems[ct_per_w:]
        wid = lax.axis_index("s") * num_cores + lax.axis_index("c")
        cb = wid * ct_per_w

        pltpu.sync_copy(
            idx_hbm.at[:, pl.ds(cb * bt, ct_per_w * bt)], idx_v
        )

        iota = lax.iota(jnp.int32, lanes)
        cvs = [
            ((iota + k * lanes) // 8) * (8 * bt) + ((iota + k * lanes) % 8) * bt
            for k in range(dim // lanes)
        ]

        def body(f, carry):
            gd = []
            for cc in range(ct_per_w):
                gd.append(
                    pltpu.async_copy(
                        table_hbm.at[idx_v.at[f, pl.ds(cc * bt, bt)]],
                        rows_v.at[cc],
                        gsems[cc],
                    )
                )
            pd = []
            for cc in range(ct_per_w):
                gd[cc].wait()

                @plsc.parallel_loop(0, bt, unroll=4)
                def transpose_body(b, _cc=cc):
                    for k in range(dim // lanes):
                        v = rows_v[_cc, b, pl.ds(k * lanes, lanes)]
                        plsc.store_scatter(ostage.at[_cc], [cvs[k] + b], v)

                for r in range(octets):
                    off = (
                        f * (octets * n_btiles * 8 * bt)
                        + r * (n_btiles * 8 * bt)
                        + (cb + cc) * (8 * bt)
                    )
                    pd.append(
                        pltpu.async_copy(
                            ostage.at[cc, pl.ds(r * 8 * bt, 8 * bt)],
                            out_hbm.at[pl.ds(off, 8 * bt)],
                            psems[cc],
                        )
                    )
            for d in pd:
                d.wait()
            return carry

        lax.fori_loop(0, fields, body, 0)

    return emb_kernel


def kernel(inputs, table):
    batch, fields = inputs.shape
    vocab, dim = table.shape
    idx_t = inputs.T.astype(jnp.int32)
    detile = _make_detiler(vocab, dim)
    table_t = table.T
    table_rm = detile(table_t, table_t[:, vocab - 128 :]).reshape(vocab, dim)
    emb = _make_kernel(batch, fields, vocab, dim)
    out1d = emb(idx_t, table_rm)
    t5 = out1d.reshape(fields, dim // 8, batch // 128, 8, 128)
    return t5.transpose(2, 4, 0, 1, 3).reshape(batch, fields, dim)
